# baseline (speedup 1.0000x reference)
import dataclasses
import functools

import jax
import jax.numpy as jnp
from jax import lax
from jax.experimental import pallas as pl
from jax.experimental.pallas import tpu as pltpu
from jax.experimental.pallas import tpu_sc as plsc


@functools.lru_cache(maxsize=None)
def _build_sc_kernel(bs, h, w, F):
    info = plsc.get_sparse_core_info()
    NC, NS, L = info.num_cores, info.num_subcores, info.num_lanes
    NW = NC * NS
    F2 = 2 * F
    FPW = F2 // NW
    assert F2 % NW == 0 and F % FPW == 0 and w % L == 0

    mesh = plsc.VectorSubcoreMesh(core_axis_name="c", subcore_axis_name="s")

    cp = pltpu.CompilerParams()
    if "needs_layout_passes" in pltpu.CompilerParams.__dataclass_fields__:
        cp = dataclasses.replace(cp, needs_layout_passes=False)

    @functools.partial(
        pl.kernel,
        mesh=mesh,
        compiler_params=cp,
        out_type=jax.ShapeDtypeStruct((bs, F2, h, w), jnp.float32),
        scratch_types=[
            pltpu.VMEM((2, max(h, w), F), jnp.float32),
            pltpu.VMEM((FPW, h, w), jnp.float32),
            pltpu.SemaphoreType.DMA,
        ],
    )
    def sc_kernel(row_hbm, col_hbm, out_hbm, tbl_v, buf_v, sem):
        wid = lax.axis_index("s") * NC + lax.axis_index("c")
        f0 = wid * FPW
        is_row = f0 >= F

        pltpu.sync_copy(col_hbm.at[pl.ds(0, w)], tbl_v.at[0, pl.ds(0, w)])
        pltpu.sync_copy(row_hbm.at[pl.ds(0, h)], tbl_v.at[1, pl.ds(0, h)])

        lanes = lax.iota(jnp.int32, L)
        t_idx = jnp.full((L,), jnp.where(is_row, 1, 0).astype(jnp.int32))
        for j in range(FPW):
            f = f0 + j
            fm = jnp.where(is_row, f - F, f).astype(jnp.int32)
            c_idx = jnp.full((L,), fm)

            @pl.loop(0, h)
            def _(y, j=j, c_idx=c_idx):
                y_splat = jnp.full((L,), y.astype(jnp.int32))
                for xh in range(w // L):
                    r_idx = jnp.where(is_row, y_splat, lanes + xh * L)
                    v = plsc.load_gather(tbl_v, [t_idx, r_idx, c_idx])
                    buf_v[j, y, pl.ds(xh * L, L)] = v

        @pl.loop(0, bs)
        def _(b):
            pltpu.async_copy(buf_v, out_hbm.at[b, pl.ds(f0, FPW)], sem)

        @pl.loop(0, bs)
        def _(b):
            pltpu.make_async_copy(
                buf_v, out_hbm.at[b, pl.ds(f0, FPW)], sem
            ).wait()

    return sc_kernel


def kernel(mask, row_embed, col_embed):
    bs, h, w = mask.shape
    F = row_embed.shape[1]
    sc_kernel = _build_sc_kernel(bs, h, w, F)
    return sc_kernel(row_embed, col_embed)


# speedup vs baseline: 3.9533x; 3.9533x over previous
import dataclasses
import functools

import jax
import jax.numpy as jnp
from jax import lax
from jax.experimental import pallas as pl
from jax.experimental.pallas import tpu as pltpu
from jax.experimental.pallas import tpu_sc as plsc


@functools.lru_cache(maxsize=None)
def _build_sc_kernel(bs, h, w, F):
    info = plsc.get_sparse_core_info()
    NC, NS, L = info.num_cores, info.num_subcores, info.num_lanes
    F2 = 2 * F
    assert h == NC * NS and F % L == 0 and bs % (2 * NS) == 0

    mesh = plsc.VectorSubcoreMesh(core_axis_name="c", subcore_axis_name="s")

    cp = pltpu.CompilerParams()
    if "needs_layout_passes" in pltpu.CompilerParams.__dataclass_fields__:
        cp = dataclasses.replace(cp, needs_layout_passes=False)

    @functools.partial(
        pl.kernel,
        mesh=mesh,
        compiler_params=cp,
        out_type=jax.ShapeDtypeStruct((bs, h, w, F2), jnp.float32),
        scratch_types=[
            pltpu.VMEM((2, max(h, w), F), jnp.float32),
            pltpu.VMEM((w, F2), jnp.float32),
            pltpu.VMEM_SHARED((NS, w, F2), jnp.float32),
            pltpu.SemaphoreType.DMA,
        ],
    )
    def sc_kernel(row_hbm, col_hbm, out_hbm, tbl_v, buf_v, half_sh, sem):
        c = lax.axis_index("c")
        s = lax.axis_index("s")
        y = c * NS + s

        pltpu.sync_copy(col_hbm.at[pl.ds(0, w)], tbl_v.at[0, pl.ds(0, w)])
        pltpu.sync_copy(row_hbm.at[pl.ds(0, h)], tbl_v.at[1, pl.ds(0, h)])

        row_vecs = [tbl_v[1, y, pl.ds(k * L, L)] for k in range(F // L)]

        @pl.loop(0, w)
        def _(x):
            for k in range(F // L):
                buf_v[x, pl.ds(k * L, L)] = tbl_v[0, x, pl.ds(k * L, L)]
                buf_v[x, pl.ds(F + k * L, L)] = row_vecs[k]

        pltpu.sync_copy(buf_v, half_sh.at[s])
        plsc.subcore_barrier()

        bpt = bs // NS
        for i in range(bpt):
            b = s * bpt + i
            pltpu.async_copy(half_sh, out_hbm.at[b, pl.ds(c * NS, NS)], sem)
        for i in range(bpt):
            b = s * bpt + i
            pltpu.make_async_copy(
                half_sh, out_hbm.at[b, pl.ds(c * NS, NS)], sem
            ).wait()

    return sc_kernel


def kernel(mask, row_embed, col_embed):
    bs, h, w = mask.shape
    F = row_embed.shape[1]
    sc_kernel = _build_sc_kernel(bs, h, w, F)
    out_bhwf = sc_kernel(row_embed, col_embed)
    return jnp.transpose(out_bhwf, (0, 3, 1, 2))


# speedup vs baseline: 13.7096x; 3.4679x over previous
import dataclasses
import functools

import jax
import jax.numpy as jnp
from jax import lax
from jax.experimental import pallas as pl
from jax.experimental.pallas import tpu as pltpu
from jax.experimental.pallas import tpu_sc as plsc


@functools.lru_cache(maxsize=None)
def _build_sc_kernel(bs, h, w, F):
    info = plsc.get_sparse_core_info()
    NC, NS, L = info.num_cores, info.num_subcores, info.num_lanes
    F2 = 2 * F
    assert h == NC * NS and F % L == 0 and bs % (2 * NS) == 0

    mesh = plsc.VectorSubcoreMesh(core_axis_name="c", subcore_axis_name="s")

    cp = pltpu.CompilerParams()
    if "needs_layout_passes" in pltpu.CompilerParams.__dataclass_fields__:
        cp = dataclasses.replace(cp, needs_layout_passes=False)

    @functools.partial(
        pl.kernel,
        mesh=mesh,
        compiler_params=cp,
        out_type=jax.ShapeDtypeStruct((bs, h, w, F2), jnp.float32),
        scratch_types=[
            pltpu.VMEM((2, max(h, w), F), jnp.float32),
            pltpu.VMEM((w, F2), jnp.float32),
            pltpu.VMEM_SHARED((NS, w, F2), jnp.float32),
            pltpu.SemaphoreType.DMA,
        ],
    )
    def sc_kernel(row_hbm, col_hbm, out_hbm, tbl_v, buf_v, half_sh, sem):
        c = lax.axis_index("c")
        s = lax.axis_index("s")
        y = c * NS + s

        pltpu.sync_copy(col_hbm.at[pl.ds(0, w)], tbl_v.at[0, pl.ds(0, w)])
        pltpu.sync_copy(row_hbm.at[pl.ds(0, h)], tbl_v.at[1, pl.ds(0, h)])

        row_vecs = [tbl_v[1, y, pl.ds(k * L, L)] for k in range(F // L)]

        @pl.loop(0, w)
        def _(x):
            for k in range(F // L):
                buf_v[x, pl.ds(k * L, L)] = tbl_v[0, x, pl.ds(k * L, L)]
                buf_v[x, pl.ds(F + k * L, L)] = row_vecs[k]

        pltpu.sync_copy(buf_v, half_sh.at[s])
        plsc.subcore_barrier()

        bpt = bs // NS
        for i in range(bpt):
            b = s * bpt + i
            pltpu.async_copy(half_sh, out_hbm.at[b, pl.ds(c * NS, NS)], sem)
        for i in range(bpt):
            b = s * bpt + i
            pltpu.make_async_copy(
                half_sh, out_hbm.at[b, pl.ds(c * NS, NS)], sem
            ).wait()

    return sc_kernel


@functools.lru_cache(maxsize=None)
def _build_tc_kernel(bs, h, w, F):
    F2 = 2 * F

    def body(row_ref, col_ref, out_ref, img_vmem, sem):
        col = col_ref[pl.ds(0, w), :]
        row = row_ref[pl.ds(0, h), :]
        img_vmem[:, :, 0:F] = jnp.broadcast_to(col[None, :, :], (h, w, F))
        img_vmem[:, :, F:F2] = jnp.broadcast_to(row[:, None, :], (h, w, F))
        for b in range(bs):
            pltpu.make_async_copy(img_vmem, out_ref.at[b], sem).start()
        for b in range(bs):
            pltpu.make_async_copy(img_vmem, out_ref.at[b], sem).wait()

    return pl.pallas_call(
        body,
        out_shape=jax.ShapeDtypeStruct((bs, h, w, F2), jnp.float32),
        in_specs=[
            pl.BlockSpec(memory_space=pltpu.VMEM),
            pl.BlockSpec(memory_space=pltpu.VMEM),
        ],
        out_specs=pl.BlockSpec(memory_space=pl.ANY),
        scratch_shapes=[
            pltpu.VMEM((h, w, F2), jnp.float32),
            pltpu.SemaphoreType.DMA,
        ],
    )


def kernel(mask, row_embed, col_embed):
    bs, h, w = mask.shape
    F = row_embed.shape[1]
    tc_kernel = _build_tc_kernel(bs, h, w, F)
    out_bhwf = tc_kernel(row_embed, col_embed)
    return jnp.transpose(out_bhwf, (0, 3, 1, 2))
